# Initial kernel scaffold; baseline (speedup 1.0000x reference)
import jax
import jax.numpy as jnp
from jax.experimental import pallas as pl


def kernel(h, coord, edge_index, coord_diff, edge_attr, W1, b1, W2, b2, W3):
    raise NotImplementedError("write your pallas kernel here")


# speedup vs baseline: 3.2881x; 3.2881x over previous
import functools

import jax
import jax.numpy as jnp
from jax import lax
from jax.experimental import pallas as pl
from jax.experimental.pallas import tpu as pltpu
from jax.experimental.pallas import tpu_sc as plsc

NC = 2
NS = 16
NW = NC * NS

COORDS_RANGE_OVER_NORM = 15.0 / 100.0


def _ab_body(h_ref, wa_ref, wb_ref, a_ref, b_ref):
    hb = h_ref[...].astype(jnp.bfloat16)
    dn = (((1,), (1,)), ((), ()))
    a_ref[...] = lax.dot_general(
        hb, wa_ref[...], dn, preferred_element_type=jnp.float32)
    b_ref[...] = lax.dot_general(
        hb, wb_ref[...], dn, preferred_element_type=jnp.float32)


def _node_ab(h, w1a, w1b):
    n, hdim = h.shape
    return pl.pallas_call(
        _ab_body,
        out_shape=(
            jax.ShapeDtypeStruct((n, hdim), jnp.float32),
            jax.ShapeDtypeStruct((n, hdim), jnp.float32),
        ),
    )(h, w1a, w1b)


IW = 80
GPC = 5
CG = IW * GPC


def _gather_pair(a_t, b_t, row1d, col1d, e):
    epw = e // NW
    mesh = plsc.VectorSubcoreMesh(core_axis_name="c", subcore_axis_name="s")
    hdim = a_t.shape[1]

    @functools.partial(
        pl.kernel,
        out_type=(
            jax.ShapeDtypeStruct((e, hdim), jnp.float32),
            jax.ShapeDtypeStruct((e, hdim), jnp.float32),
        ),
        mesh=mesh,
        scratch_types=[
            pltpu.VMEM((CG,), jnp.int32),
            pltpu.VMEM((CG,), jnp.int32),
            pltpu.VMEM((CG, hdim), jnp.float32),
            pltpu.VMEM((CG, hdim), jnp.float32),
            pltpu.SemaphoreType.DMA,
            pltpu.SemaphoreType.DMA,
        ],
    )
    def k(a_hbm, b_hbm, row_hbm, col_hbm, ga_hbm, gb_hbm,
          rowv, colv, bufa, bufb, sema, semb):
        wid = lax.axis_index("c") * NS + lax.axis_index("s")

        def chunk(i, carry):
            base = wid * epw + i * CG
            pltpu.sync_copy(row_hbm.at[pl.ds(base, CG)], rowv)
            pltpu.sync_copy(col_hbm.at[pl.ds(base, CG)], colv)
            descs = []
            for j in range(GPC):
                descs.append(pltpu.async_copy(
                    a_hbm.at[rowv.at[pl.ds(j * IW, IW)]],
                    bufa.at[pl.ds(j * IW, IW)], sema))
                descs.append(pltpu.async_copy(
                    b_hbm.at[colv.at[pl.ds(j * IW, IW)]],
                    bufb.at[pl.ds(j * IW, IW)], semb))
            for d in descs:
                d.wait()
            pltpu.sync_copy(bufa, ga_hbm.at[pl.ds(base, CG)])
            pltpu.sync_copy(bufb, gb_hbm.at[pl.ds(base, CG)])
            return carry

        lax.fori_loop(0, epw // CG, chunk, 0)

    return k(a_t, b_t, row1d, col1d)


BE = 512


def _mlp_body(ga_ref, gb_ref, ea_ref, cd_ref, w1c_ref, b1_ref, w2_ref,
              b2_ref, w3_ref, out_ref):
    dn = (((1,), (1,)), ((), ()))
    pre = ga_ref[...] + gb_ref[...]
    pre = pre + lax.dot_general(
        ea_ref[...].astype(jnp.bfloat16), w1c_ref[...], dn,
        preferred_element_type=jnp.float32)
    pre = pre + b1_ref[...]
    x = (pre * jax.nn.sigmoid(pre)).astype(jnp.bfloat16)
    pre2 = lax.dot_general(
        x, w2_ref[...], dn, preferred_element_type=jnp.float32) + b2_ref[...]
    y = (pre2 * jax.nn.sigmoid(pre2)).astype(jnp.bfloat16)
    s = lax.dot_general(
        w3_ref[...], y, dn, preferred_element_type=jnp.float32)
    tr = cd_ref[...] * (jnp.tanh(s) * COORDS_RANGE_OVER_NORM)
    out_ref[0][...] = tr[0]
    out_ref[1][...] = tr[1]
    out_ref[2][...] = tr[2]


def _edge_mlp(ga, gb, ea, cdt, w1c, b1r, w2, b2r, w3):
    e, hdim = ga.shape
    de = ea.shape[1]
    grid = (e // BE,)

    def body(ga_ref, gb_ref, ea_ref, cd_ref, w1c_ref, b1_ref, w2_ref,
             b2_ref, w3_ref, o0_ref, o1_ref, o2_ref):
        _mlp_body(ga_ref, gb_ref, ea_ref, cd_ref, w1c_ref, b1_ref, w2_ref,
                  b2_ref, w3_ref, (o0_ref, o1_ref, o2_ref))

    return pl.pallas_call(
        body,
        grid=grid,
        in_specs=[
            pl.BlockSpec((BE, hdim), lambda i: (i, 0)),
            pl.BlockSpec((BE, hdim), lambda i: (i, 0)),
            pl.BlockSpec((BE, de), lambda i: (i, 0)),
            pl.BlockSpec((3, BE), lambda i: (0, i)),
            pl.BlockSpec((hdim, de), lambda i: (0, 0)),
            pl.BlockSpec((1, hdim), lambda i: (0, 0)),
            pl.BlockSpec((hdim, hdim), lambda i: (0, 0)),
            pl.BlockSpec((1, hdim), lambda i: (0, 0)),
            pl.BlockSpec((1, hdim), lambda i: (0, 0)),
        ],
        out_specs=[
            pl.BlockSpec((BE,), lambda i: (i,)),
            pl.BlockSpec((BE,), lambda i: (i,)),
            pl.BlockSpec((BE,), lambda i: (i,)),
        ],
        out_shape=[
            jax.ShapeDtypeStruct((e,), jnp.float32),
            jax.ShapeDtypeStruct((e,), jnp.float32),
            jax.ShapeDtypeStruct((e,), jnp.float32),
        ],
    )(ga, gb, ea, cdt, w1c, b1r, w2, b2r, w3)


ACC = 32768
C4 = 2000


def _segment_scatter(tr0, tr1, tr2, row1d, zeros1d, e):
    epw = e // NW
    mesh = plsc.VectorSubcoreMesh(core_axis_name="c", subcore_axis_name="s")
    sl = ACC // NS

    @functools.partial(
        pl.kernel,
        out_type=jax.ShapeDtypeStruct((NC, ACC // 128, 128), jnp.float32),
        mesh=mesh,
        scratch_types=[
            pltpu.VMEM((C4,), jnp.int32),
            pltpu.VMEM((C4,), jnp.float32),
            pltpu.VMEM((C4,), jnp.float32),
            pltpu.VMEM((C4,), jnp.float32),
            pltpu.VMEM((ACC,), jnp.float32),
            pltpu.VMEM((NS, sl), jnp.float32),
            pltpu.VMEM((sl // 128, 128), jnp.float32),
            pltpu.VMEM_SHARED((NS, ACC), jnp.float32),
        ],
        compiler_params=pltpu.CompilerParams(needs_layout_passes=False),
    )
    def k(tr0_hbm, tr1_hbm, tr2_hbm, row_hbm, zero_hbm, out_hbm,
          rowv, t0, t1, t2, accl, buf2, res, stage):
        cid = lax.axis_index("c")
        sid = lax.axis_index("s")
        wid = cid * NS + sid

        pltpu.sync_copy(zero_hbm, accl)

        def chunk(i, carry):
            base = wid * epw + i * C4
            pltpu.sync_copy(row_hbm.at[pl.ds(base, C4)], rowv)
            pltpu.sync_copy(tr0_hbm.at[pl.ds(base, C4)], t0)
            pltpu.sync_copy(tr1_hbm.at[pl.ds(base, C4)], t1)
            pltpu.sync_copy(tr2_hbm.at[pl.ds(base, C4)], t2)

            def grp(g, c2):
                rv = rowv[pl.ds(g * 16, 16)]
                f0 = rv * 3
                for d, tref in enumerate((t0, t1, t2)):
                    plsc.addupdate_scatter(
                        accl, [f0 + d], tref[pl.ds(g * 16, 16)])
                return c2

            lax.fori_loop(0, C4 // 16, grp, 0)
            return carry

        lax.fori_loop(0, epw // C4, chunk, 0)

        pltpu.sync_copy(accl, stage.at[sid])
        plsc.subcore_barrier()
        pltpu.sync_copy(stage.at[:, pl.ds(sid * sl, sl)], buf2)

        def colsum_rows(jr, carry):
            for jc in range(8):
                j = jr * 8 + jc
                acc16 = buf2[0, pl.ds(j * 16, 16)]
                for r in range(1, NS):
                    acc16 = acc16 + buf2[r, pl.ds(j * 16, 16)]
                res[jr, pl.ds(jc * 16, 16)] = acc16
            return carry

        lax.fori_loop(0, sl // 128, colsum_rows, 0)
        pltpu.sync_copy(res, out_hbm.at[cid, pl.ds(sid * (sl // 128),
                                                   sl // 128)])

    return k(tr0, tr1, tr2, row1d, zeros1d)


def kernel(h, coord, edge_index, coord_diff, edge_attr, W1, b1, W2, b2, W3):
    n, hdim = h.shape
    e = edge_index.shape[1]

    w1a = W1[:, :hdim].astype(jnp.bfloat16)
    w1b = W1[:, hdim:2 * hdim].astype(jnp.bfloat16)
    w1c = W1[:, 2 * hdim:].astype(jnp.bfloat16)

    a_t, b_t = _node_ab(h, w1a, w1b)

    row = edge_index[0]
    col = edge_index[1]
    ga, gb = _gather_pair(a_t, b_t, row, col, e)

    tr0, tr1, tr2 = _edge_mlp(
        ga, gb, edge_attr, coord_diff.T, w1c,
        b1.reshape(1, -1), W2.astype(jnp.bfloat16), b2.reshape(1, -1),
        W3.astype(jnp.bfloat16))

    zeros1d = jnp.zeros((ACC,), dtype=jnp.float32)
    partials = _segment_scatter(tr0, tr1, tr2, row, zeros1d, e)

    agg = (partials[0] + partials[1]).reshape(-1)[:3 * n].reshape(n, 3)
    return coord + agg
